# Initial kernel scaffold; baseline (speedup 1.0000x reference)
import jax
import jax.numpy as jnp
from jax.experimental import pallas as pl


def kernel(u, v, u_table, v_table, gcl_W, gcl_b, dense_W, dense_b, bilin_W, ratings):
    raise NotImplementedError("write your pallas kernel here")


# speedup vs baseline: 1.3826x; 1.3826x over previous
import functools

import jax
import jax.numpy as jnp
from jax import lax
from jax.experimental import pallas as pl
from jax.experimental.pallas import tpu as pltpu
from jax.experimental.pallas import tpu_sc as plsc

_R = 5
_NU = 10000
_NV = 1000
_D = 128
_H0 = 64
_H1 = 32
_BU = 4096
_BV = 1000

_NC = 2
_NS = 16
_NW = _NC * _NS
_ROWS_W = (_R * _BU) // _NW
_CHUNK = 40
_NCHUNK = _ROWS_W // _CHUNK
_UE_W = _BU // _NW
_BVP = 1024
_VE_W = _BVP // _NW

_BU3 = 256
_BU5 = 256


def _sc_gather_body(ratings2d, ridx3, u2, v2, u_table, v_table,
                    r_out, ue_out, ve_out,
                    idx2, uidx, vidx, rows_a, rows_b, erows, vrows,
                    sem_a, sem_b, sem_e):
    wid = lax.axis_index("s") * _NC + lax.axis_index("c")
    base = wid * _ROWS_W

    pltpu.sync_copy(ridx3.at[wid], idx2)

    pltpu.sync_copy(u2.at[wid], uidx)
    pltpu.async_copy(u_table.at[uidx], erows, sem_e).wait()
    pltpu.sync_copy(erows, ue_out.at[pl.ds(wid * _UE_W, _UE_W)])

    pltpu.sync_copy(v2.at[wid], vidx)
    pltpu.async_copy(v_table.at[vidx], vrows, sem_e).wait()
    pltpu.sync_copy(vrows, ve_out.at[pl.ds(wid * _VE_W, _VE_W)])

    bufs = (rows_a, rows_b)
    sems = (sem_a, sem_b)
    handles = [None, None]
    handles[0] = pltpu.async_copy(ratings2d.at[idx2.at[0]], rows_a, sem_a)
    for c in range(_NCHUNK):
        if c + 1 < _NCHUNK:
            handles[(c + 1) % 2] = pltpu.async_copy(
                ratings2d.at[idx2.at[c + 1]], bufs[(c + 1) % 2],
                sems[(c + 1) % 2])
        handles[c % 2].wait()
        pltpu.sync_copy(bufs[c % 2],
                        r_out.at[pl.ds(base + c * _CHUNK, _CHUNK)])


_sc_gather = functools.partial(
    pl.kernel,
    mesh=plsc.VectorSubcoreMesh(core_axis_name="c", subcore_axis_name="s"),
    out_type=[
        jax.ShapeDtypeStruct((_R * _BU, _BV), jnp.float32),
        jax.ShapeDtypeStruct((_BU, _D), jnp.float32),
        jax.ShapeDtypeStruct((_BVP, _D), jnp.float32),
    ],
    scratch_types=[
        pltpu.VMEM((_NCHUNK, _CHUNK), jnp.int32),
        pltpu.VMEM((_UE_W,), jnp.int32),
        pltpu.VMEM((_VE_W,), jnp.int32),
        pltpu.VMEM((_CHUNK, _BV), jnp.float32),
        pltpu.VMEM((_CHUNK, _BV), jnp.float32),
        pltpu.VMEM((_UE_W, _D), jnp.float32),
        pltpu.VMEM((_VE_W, _D), jnp.float32),
        pltpu.SemaphoreType.DMA,
        pltpu.SemaphoreType.DMA,
        pltpu.SemaphoreType.DMA,
    ],
    compiler_params=pltpu.CompilerParams(use_tc_tiling_on_sc=False),
)(_sc_gather_body)


def _prep_body(vemb_ref, gclW_ref, v_ref, bv_ref, cnt_ref):
    kio = lax.broadcasted_iota(jnp.int32, (_NV, _BV), 0)
    G = (kio == v_ref[...]).astype(jnp.float32)
    cnt_ref[...] = jnp.sum(G, axis=1, keepdims=True)
    ve = vemb_ref[...]
    for r in range(_R):
        sv = jnp.dot(ve, gclW_ref[r], preferred_element_type=jnp.float32)
        bv_ref[r] = jnp.dot(G, sv, preferred_element_type=jnp.float32)


def _prep_call(vemb, gcl_W, v_row):
    return pl.pallas_call(
        _prep_body,
        out_shape=[
            jax.ShapeDtypeStruct((_R, _NV, _H0), jnp.float32),
            jax.ShapeDtypeStruct((_NV, 1), jnp.float32),
        ],
    )(vemb, gcl_W, v_row)


def _agg_body(r3_ref, ue_ref, gclW_ref, bv_ref, cnt_ref, v_ref,
              msgu_ref, di_ref, cs_ref, tv_ref, tcode_ref):
    i = pl.program_id(0)
    ue = ue_ref[...]
    acc_msg = jnp.zeros((_BU3, _H0), jnp.float32)
    acc_di = jnp.zeros((_BU3, 1), jnp.float32)
    acc_cs = jnp.zeros((1, _NV), jnp.float32)
    acc_tv = jnp.zeros((_NV, _H0), jnp.float32)
    tc0 = jnp.zeros((_BU3, _NV), jnp.float32)
    for r in range(_R):
        Rr = r3_ref[r]
        su = jnp.dot(ue, gclW_ref[r], preferred_element_type=jnp.float32)
        acc_msg = acc_msg + jnp.dot(Rr, bv_ref[r],
                                    preferred_element_type=jnp.float32)
        acc_di = acc_di + jnp.dot(Rr, cnt_ref[...],
                                  preferred_element_type=jnp.float32)
        acc_cs = acc_cs + jnp.sum(Rr, axis=0, keepdims=True)
        acc_tv = acc_tv + lax.dot_general(
            Rr, su, (((0,), (0,)), ((), ())),
            preferred_element_type=jnp.float32)
        tc0 = tc0 + float(r + 1) * Rr

    kio = lax.broadcasted_iota(jnp.int32, (_NV, _BV), 0)
    Gb = (kio == v_ref[...]).astype(jnp.bfloat16)
    tcode = lax.dot_general(
        tc0.astype(jnp.bfloat16), Gb, (((1,), (0,)), ((), ())),
        preferred_element_type=jnp.float32)
    tcode_ref[...] = tcode.astype(jnp.bfloat16)
    msgu_ref[...] = acc_msg
    di_ref[...] = acc_di

    @pl.when(i == 0)
    def _():
        cs_ref[...] = acc_cs
        tv_ref[...] = acc_tv

    @pl.when(i != 0)
    def _():
        cs_ref[...] = cs_ref[...] + acc_cs
        tv_ref[...] = tv_ref[...] + acc_tv


def _agg_call(r3, uemb, gcl_W, bv, cnt, v_row):
    n = _BU // _BU3
    return pl.pallas_call(
        _agg_body,
        grid=(n,),
        in_specs=[
            pl.BlockSpec((_R, _BU3, _BV), lambda i: (0, i, 0)),
            pl.BlockSpec((_BU3, _D), lambda i: (i, 0)),
            pl.BlockSpec((_R, _D, _H0), lambda i: (0, 0, 0)),
            pl.BlockSpec((_R, _NV, _H0), lambda i: (0, 0, 0)),
            pl.BlockSpec((_NV, 1), lambda i: (0, 0)),
            pl.BlockSpec((1, _BV), lambda i: (0, 0)),
        ],
        out_specs=[
            pl.BlockSpec((_BU3, _H0), lambda i: (i, 0)),
            pl.BlockSpec((_BU3, 1), lambda i: (i, 0)),
            pl.BlockSpec((1, _NV), lambda i: (0, 0)),
            pl.BlockSpec((_NV, _H0), lambda i: (0, 0)),
            pl.BlockSpec((_BU3, _BV), lambda i: (i, 0)),
        ],
        out_shape=[
            jax.ShapeDtypeStruct((_BU, _H0), jnp.float32),
            jax.ShapeDtypeStruct((_BU, 1), jnp.float32),
            jax.ShapeDtypeStruct((1, _NV), jnp.float32),
            jax.ShapeDtypeStruct((_NV, _H0), jnp.float32),
            jax.ShapeDtypeStruct((_BU, _BV), jnp.bfloat16),
        ],
        compiler_params=pltpu.CompilerParams(
            dimension_semantics=("arbitrary",)),
    )(r3, uemb, gcl_W, bv, cnt, v_row)


def _hidden_body(msgu_ref, di_ref, cs_ref, tv_ref, v_ref, dW_ref, db_ref,
                 gclb_ref, uh_ref, vh_ref):
    kio = lax.broadcasted_iota(jnp.int32, (_NV, _BV), 0)
    G = (kio == v_ref[...]).astype(jnp.float32)
    du = lax.dot_general(G, cs_ref[...], (((0,), (1,)), ((), ())),
                         preferred_element_type=jnp.float32)
    msgv = lax.dot_general(G, tv_ref[...], (((0,), (0,)), ((), ())),
                           preferred_element_type=jnp.float32)
    deg = jnp.concatenate([du, di_ref[...]], axis=0)
    c = jnp.where(deg > 0, 1.0 / jnp.where(deg > 0, deg, 1.0), 0.0)
    cu = c[:_BU]
    ci = c[_BU:]
    bsum = jnp.sum(gclb_ref[...], axis=0, keepdims=True)
    zu = jnp.maximum(msgu_ref[...] * cu + bsum, 0.0)
    zv = jnp.maximum(msgv * ci + bsum, 0.0)
    dW = dW_ref[...]
    db = db_ref[...]
    uh_ref[...] = jax.nn.sigmoid(
        jnp.dot(zu, dW, preferred_element_type=jnp.float32) + db)
    vh_ref[...] = jax.nn.sigmoid(
        jnp.dot(zv, dW, preferred_element_type=jnp.float32) + db)


def _hidden_call(msgu, di, cs, tv, v_row, dense_W, db_row, gcl_b):
    return pl.pallas_call(
        _hidden_body,
        out_shape=[
            jax.ShapeDtypeStruct((_BU, _H1), jnp.float32),
            jax.ShapeDtypeStruct((_BV, _H1), jnp.float32),
        ],
    )(msgu, di, cs, tv, v_row, dense_W, db_row, gcl_b)


def _dec_body(uh_ref, vh_ref, bw_ref, tcode_ref,
              mhat_ref, loss_ref, acc_ref, sacc):
    i = pl.program_id(0)
    uh = uh_ref[...]
    vh = vh_ref[...]
    Os = []
    for r in range(_R):
        A = jnp.dot(uh, bw_ref[r], preferred_element_type=jnp.float32)
        Os.append(lax.dot_general(A, vh, (((1,), (1,)), ((), ())),
                                  preferred_element_type=jnp.float32))
    mx = Os[0]
    for r in range(1, _R):
        mx = jnp.maximum(mx, Os[r])
    es = [jnp.exp(o - mx) for o in Os]
    se = es[0]
    for r in range(1, _R):
        se = se + es[r]
    num = jnp.zeros_like(se)
    for r in range(1, _R):
        num = num + float(r) * es[r]
    mhat_ref[...] = num / se

    tc = tcode_ref[...].astype(jnp.float32)
    obs = tc > 0.5
    ot = jnp.zeros_like(mx)
    for r in range(_R):
        ot = jnp.where(tc == float(r + 1), Os[r], ot)
    lterm = jnp.where(obs, mx + jnp.log(se) - ot, 0.0)

    pbest = Os[0]
    pcls = jnp.zeros_like(mx)
    for r in range(1, _R):
        gt = Os[r] > pbest
        pbest = jnp.where(gt, Os[r], pbest)
        pcls = jnp.where(gt, float(r), pcls)
    corr = jnp.where(obs & (pcls == (tc - 1.0)), 1.0, 0.0)

    ls = jnp.sum(lterm)
    nb = jnp.sum(jnp.where(obs, 1.0, 0.0))
    cr = jnp.sum(corr)

    @pl.when(i == 0)
    def _():
        sacc[0] = ls
        sacc[1] = nb
        sacc[2] = cr

    @pl.when(i != 0)
    def _():
        sacc[0] = sacc[0] + ls
        sacc[1] = sacc[1] + nb
        sacc[2] = sacc[2] + cr

    @pl.when(i == pl.num_programs(0) - 1)
    def _():
        nbm = jnp.maximum(sacc[1], 1.0)
        loss_ref[...] = jnp.broadcast_to(sacc[0] / nbm, (1, 1))
        acc_ref[...] = jnp.broadcast_to(sacc[2] / nbm, (1, 1))


def _dec_call(uh, vh, bilin_W, tcode):
    n = _BU // _BU5
    return pl.pallas_call(
        _dec_body,
        grid=(n,),
        in_specs=[
            pl.BlockSpec((_BU5, _H1), lambda i: (i, 0)),
            pl.BlockSpec((_BV, _H1), lambda i: (0, 0)),
            pl.BlockSpec((_R, _H1, _H1), lambda i: (0, 0, 0)),
            pl.BlockSpec((_BU5, _BV), lambda i: (i, 0)),
        ],
        out_specs=[
            pl.BlockSpec((_BU5, _BV), lambda i: (i, 0)),
            pl.BlockSpec((1, 1), lambda i: (0, 0)),
            pl.BlockSpec((1, 1), lambda i: (0, 0)),
        ],
        out_shape=[
            jax.ShapeDtypeStruct((_BU, _BV), jnp.float32),
            jax.ShapeDtypeStruct((1, 1), jnp.float32),
            jax.ShapeDtypeStruct((1, 1), jnp.float32),
        ],
        scratch_shapes=[pltpu.SMEM((3,), jnp.float32)],
        compiler_params=pltpu.CompilerParams(
            dimension_semantics=("arbitrary",)),
    )(uh, vh, bilin_W, tcode)


def kernel(u, v, u_table, v_table, gcl_W, gcl_b, dense_W, dense_b, bilin_W,
           ratings):
    u = u.astype(jnp.int32)
    v = v.astype(jnp.int32)
    ratings2d = ratings.reshape(_R * _NU, _NV)
    ridx3 = (u[None, :] + (_NU * jnp.arange(_R, dtype=jnp.int32))[:, None]
             ).reshape(_NW, _NCHUNK, _CHUNK)
    u2 = u.reshape(_NW, _UE_W)
    v2 = jnp.concatenate([v, jnp.zeros((_BVP - _BV,), jnp.int32)]
                         ).reshape(_NW, _VE_W)

    r_g, uemb, vemb_p = _sc_gather(ratings2d, ridx3, u2, v2, u_table, v_table)
    vemb = vemb_p[:_NV]
    r3 = r_g.reshape(_R, _BU, _NV)
    v_row = v.reshape(1, _BV)

    bv, cnt = _prep_call(vemb, gcl_W, v_row)
    msgu, di, cs, tv, tcode = _agg_call(r3, uemb, gcl_W, bv, cnt, v_row)
    uh, vh = _hidden_call(msgu, di, cs, tv, v_row, dense_W,
                          dense_b.reshape(1, _H1), gcl_b)
    mhat, loss, acc = _dec_call(uh, vh, bilin_W, tcode)
    return mhat, loss[0, 0], acc[0, 0]


# speedup vs baseline: 4.4200x; 3.1968x over previous
import functools

import jax
import jax.numpy as jnp
from jax import lax
from jax.experimental import pallas as pl
from jax.experimental.pallas import tpu as pltpu
from jax.experimental.pallas import tpu_sc as plsc

_R = 5
_NU = 10000
_NV = 1000
_D = 128
_H0 = 64
_H1 = 32
_BU = 4096
_BV = 1000
_KP = 1024

_NC = 2
_NS = 16
_NW = _NC * _NS
_GR_W = _BU // _NW
_CHUNK = 32
_NCHUNK = _GR_W // _CHUNK
_BVP = 1024
_VE_W = _BVP // _NW

_BUC = 400
_BU3 = 256
_BU5 = 256


def _compress_body(ratings_ref, code_ref):
    acc = ratings_ref[0]
    for r in range(1, _R):
        acc = acc + float(r + 1) * ratings_ref[r]
    code_ref[...] = jnp.zeros((_BUC, _KP), jnp.float32)
    code_ref[:, : _NV] = acc


def _compress_call(ratings):
    n = _NU // _BUC
    return pl.pallas_call(
        _compress_body,
        grid=(n,),
        in_specs=[pl.BlockSpec((_R, _BUC, _NV), lambda i: (0, i, 0))],
        out_specs=pl.BlockSpec((_BUC, _KP), lambda i: (i, 0)),
        out_shape=jax.ShapeDtypeStruct((_NU, _KP), jnp.float32),
        compiler_params=pltpu.CompilerParams(
            dimension_semantics=("arbitrary",)),
    )(ratings)


def _sc_gather_body(code, u2, v2, u_table, v_table,
                    gc_out, ue_out, ve_out,
                    uidx, vidx, rows_a, rows_b, erows, vrows,
                    sem_a, sem_b, sem_e):
    wid = lax.axis_index("s") * _NC + lax.axis_index("c")
    base = wid * _GR_W

    pltpu.sync_copy(u2.at[wid], uidx)

    pltpu.async_copy(u_table.at[uidx], erows, sem_e).wait()
    pltpu.sync_copy(erows, ue_out.at[pl.ds(base, _GR_W)])

    pltpu.sync_copy(v2.at[wid], vidx)
    pltpu.async_copy(v_table.at[vidx], vrows, sem_e).wait()
    pltpu.sync_copy(vrows, ve_out.at[pl.ds(wid * _VE_W, _VE_W)])

    bufs = (rows_a, rows_b)
    sems = (sem_a, sem_b)
    handles = [None, None]
    handles[0] = pltpu.async_copy(
        code.at[uidx.at[pl.ds(0, _CHUNK)]], rows_a, sem_a)
    for c in range(_NCHUNK):
        if c + 1 < _NCHUNK:
            handles[(c + 1) % 2] = pltpu.async_copy(
                code.at[uidx.at[pl.ds((c + 1) * _CHUNK, _CHUNK)]],
                bufs[(c + 1) % 2], sems[(c + 1) % 2])
        handles[c % 2].wait()
        pltpu.sync_copy(bufs[c % 2],
                        gc_out.at[pl.ds(base + c * _CHUNK, _CHUNK)])


_sc_gather = functools.partial(
    pl.kernel,
    mesh=plsc.VectorSubcoreMesh(core_axis_name="c", subcore_axis_name="s"),
    out_type=[
        jax.ShapeDtypeStruct((_BU, _KP), jnp.float32),
        jax.ShapeDtypeStruct((_BU, _D), jnp.float32),
        jax.ShapeDtypeStruct((_BVP, _D), jnp.float32),
    ],
    scratch_types=[
        pltpu.VMEM((_GR_W,), jnp.int32),
        pltpu.VMEM((_VE_W,), jnp.int32),
        pltpu.VMEM((_CHUNK, _KP), jnp.float32),
        pltpu.VMEM((_CHUNK, _KP), jnp.float32),
        pltpu.VMEM((_GR_W, _D), jnp.float32),
        pltpu.VMEM((_VE_W, _D), jnp.float32),
        pltpu.SemaphoreType.DMA,
        pltpu.SemaphoreType.DMA,
        pltpu.SemaphoreType.DMA,
    ],
)(_sc_gather_body)


def _prep_body(vemb_ref, gclW_ref, v_ref, bv_ref, cnt_ref):
    kio = lax.broadcasted_iota(jnp.int32, (_KP, _BV), 0)
    G = (kio == v_ref[...]).astype(jnp.float32)
    cnt_ref[...] = jnp.sum(G, axis=1, keepdims=True)
    ve = vemb_ref[...]
    for r in range(_R):
        sv = jnp.dot(ve, gclW_ref[r], preferred_element_type=jnp.float32)
        bv_ref[r] = jnp.dot(G, sv, preferred_element_type=jnp.float32)


def _prep_call(vemb, gcl_W, v_row):
    return pl.pallas_call(
        _prep_body,
        out_shape=[
            jax.ShapeDtypeStruct((_R, _KP, _H0), jnp.float32),
            jax.ShapeDtypeStruct((_KP, 1), jnp.float32),
        ],
    )(vemb, gcl_W, v_row)


def _agg_body(gc_ref, ue_ref, gclW_ref, bv_ref, cnt_ref, v_ref,
              msgu_ref, di_ref, cs_ref, tv_ref, tcode_ref):
    i = pl.program_id(0)
    ue = ue_ref[...]
    gc = gc_ref[...]
    acc_msg = jnp.zeros((_BU3, _H0), jnp.float32)
    acc_di = jnp.zeros((_BU3, 1), jnp.float32)
    acc_cs = jnp.zeros((1, _KP), jnp.float32)
    acc_tv = jnp.zeros((_KP, _H0), jnp.float32)
    for r in range(_R):
        Rr = (gc == float(r + 1)).astype(jnp.float32)
        su = jnp.dot(ue, gclW_ref[r], preferred_element_type=jnp.float32)
        acc_msg = acc_msg + jnp.dot(Rr, bv_ref[r],
                                    preferred_element_type=jnp.float32)
        acc_di = acc_di + jnp.dot(Rr, cnt_ref[...],
                                  preferred_element_type=jnp.float32)
        acc_cs = acc_cs + jnp.sum(Rr, axis=0, keepdims=True)
        acc_tv = acc_tv + lax.dot_general(
            Rr, su, (((0,), (0,)), ((), ())),
            preferred_element_type=jnp.float32)

    kio = lax.broadcasted_iota(jnp.int32, (_KP, _BV), 0)
    Gb = (kio == v_ref[...]).astype(jnp.bfloat16)
    tcode = lax.dot_general(
        gc.astype(jnp.bfloat16), Gb, (((1,), (0,)), ((), ())),
        preferred_element_type=jnp.float32)
    tcode_ref[...] = tcode.astype(jnp.bfloat16)
    msgu_ref[...] = acc_msg
    di_ref[...] = acc_di

    @pl.when(i == 0)
    def _():
        cs_ref[...] = acc_cs
        tv_ref[...] = acc_tv

    @pl.when(i != 0)
    def _():
        cs_ref[...] = cs_ref[...] + acc_cs
        tv_ref[...] = tv_ref[...] + acc_tv


def _agg_call(gcode, uemb, gcl_W, bv, cnt, v_row):
    n = _BU // _BU3
    return pl.pallas_call(
        _agg_body,
        grid=(n,),
        in_specs=[
            pl.BlockSpec((_BU3, _KP), lambda i: (i, 0)),
            pl.BlockSpec((_BU3, _D), lambda i: (i, 0)),
            pl.BlockSpec((_R, _D, _H0), lambda i: (0, 0, 0)),
            pl.BlockSpec((_R, _KP, _H0), lambda i: (0, 0, 0)),
            pl.BlockSpec((_KP, 1), lambda i: (0, 0)),
            pl.BlockSpec((1, _BV), lambda i: (0, 0)),
        ],
        out_specs=[
            pl.BlockSpec((_BU3, _H0), lambda i: (i, 0)),
            pl.BlockSpec((_BU3, 1), lambda i: (i, 0)),
            pl.BlockSpec((1, _KP), lambda i: (0, 0)),
            pl.BlockSpec((_KP, _H0), lambda i: (0, 0)),
            pl.BlockSpec((_BU3, _BV), lambda i: (i, 0)),
        ],
        out_shape=[
            jax.ShapeDtypeStruct((_BU, _H0), jnp.float32),
            jax.ShapeDtypeStruct((_BU, 1), jnp.float32),
            jax.ShapeDtypeStruct((1, _KP), jnp.float32),
            jax.ShapeDtypeStruct((_KP, _H0), jnp.float32),
            jax.ShapeDtypeStruct((_BU, _BV), jnp.bfloat16),
        ],
        compiler_params=pltpu.CompilerParams(
            dimension_semantics=("arbitrary",)),
    )(gcode, uemb, gcl_W, bv, cnt, v_row)


def _hidden_body(msgu_ref, di_ref, cs_ref, tv_ref, v_ref, dW_ref, db_ref,
                 gclb_ref, uh_ref, vh_ref):
    kio = lax.broadcasted_iota(jnp.int32, (_KP, _BV), 0)
    G = (kio == v_ref[...]).astype(jnp.float32)
    du = lax.dot_general(G, cs_ref[...], (((0,), (1,)), ((), ())),
                         preferred_element_type=jnp.float32)
    msgv = lax.dot_general(G, tv_ref[...], (((0,), (0,)), ((), ())),
                           preferred_element_type=jnp.float32)
    deg = jnp.concatenate([du, di_ref[...]], axis=0)
    c = jnp.where(deg > 0, 1.0 / jnp.where(deg > 0, deg, 1.0), 0.0)
    cu = c[:_BU]
    ci = c[_BU:]
    bsum = jnp.sum(gclb_ref[...], axis=0, keepdims=True)
    zu = jnp.maximum(msgu_ref[...] * cu + bsum, 0.0)
    zv = jnp.maximum(msgv * ci + bsum, 0.0)
    dW = dW_ref[...]
    db = db_ref[...]
    uh_ref[...] = jax.nn.sigmoid(
        jnp.dot(zu, dW, preferred_element_type=jnp.float32) + db)
    vh_ref[...] = jax.nn.sigmoid(
        jnp.dot(zv, dW, preferred_element_type=jnp.float32) + db)


def _hidden_call(msgu, di, cs, tv, v_row, dense_W, db_row, gcl_b):
    return pl.pallas_call(
        _hidden_body,
        out_shape=[
            jax.ShapeDtypeStruct((_BU, _H1), jnp.float32),
            jax.ShapeDtypeStruct((_BV, _H1), jnp.float32),
        ],
    )(msgu, di, cs, tv, v_row, dense_W, db_row, gcl_b)


def _dec_body(uh_ref, vh_ref, bw_ref, tcode_ref,
              mhat_ref, loss_ref, acc_ref, sacc):
    i = pl.program_id(0)
    uh = uh_ref[...]
    vh = vh_ref[...]
    Os = []
    for r in range(_R):
        A = jnp.dot(uh, bw_ref[r], preferred_element_type=jnp.float32)
        Os.append(lax.dot_general(A, vh, (((1,), (1,)), ((), ())),
                                  preferred_element_type=jnp.float32))
    mx = Os[0]
    for r in range(1, _R):
        mx = jnp.maximum(mx, Os[r])
    es = [jnp.exp(o - mx) for o in Os]
    se = es[0]
    for r in range(1, _R):
        se = se + es[r]
    num = jnp.zeros_like(se)
    for r in range(1, _R):
        num = num + float(r) * es[r]
    mhat_ref[...] = num / se

    tc = tcode_ref[...].astype(jnp.float32)
    obs = tc > 0.5
    ot = jnp.zeros_like(mx)
    for r in range(_R):
        ot = jnp.where(tc == float(r + 1), Os[r], ot)
    lterm = jnp.where(obs, mx + jnp.log(se) - ot, 0.0)

    pbest = Os[0]
    pcls = jnp.zeros_like(mx)
    for r in range(1, _R):
        gt = Os[r] > pbest
        pbest = jnp.where(gt, Os[r], pbest)
        pcls = jnp.where(gt, float(r), pcls)
    corr = jnp.where(obs & (pcls == (tc - 1.0)), 1.0, 0.0)

    ls = jnp.sum(lterm)
    nb = jnp.sum(jnp.where(obs, 1.0, 0.0))
    cr = jnp.sum(corr)

    @pl.when(i == 0)
    def _():
        sacc[0] = ls
        sacc[1] = nb
        sacc[2] = cr

    @pl.when(i != 0)
    def _():
        sacc[0] = sacc[0] + ls
        sacc[1] = sacc[1] + nb
        sacc[2] = sacc[2] + cr

    @pl.when(i == pl.num_programs(0) - 1)
    def _():
        nbm = jnp.maximum(sacc[1], 1.0)
        loss_ref[...] = jnp.broadcast_to(sacc[0] / nbm, (1, 1))
        acc_ref[...] = jnp.broadcast_to(sacc[2] / nbm, (1, 1))


def _dec_call(uh, vh, bilin_W, tcode):
    n = _BU // _BU5
    return pl.pallas_call(
        _dec_body,
        grid=(n,),
        in_specs=[
            pl.BlockSpec((_BU5, _H1), lambda i: (i, 0)),
            pl.BlockSpec((_BV, _H1), lambda i: (0, 0)),
            pl.BlockSpec((_R, _H1, _H1), lambda i: (0, 0, 0)),
            pl.BlockSpec((_BU5, _BV), lambda i: (i, 0)),
        ],
        out_specs=[
            pl.BlockSpec((_BU5, _BV), lambda i: (i, 0)),
            pl.BlockSpec((1, 1), lambda i: (0, 0)),
            pl.BlockSpec((1, 1), lambda i: (0, 0)),
        ],
        out_shape=[
            jax.ShapeDtypeStruct((_BU, _BV), jnp.float32),
            jax.ShapeDtypeStruct((1, 1), jnp.float32),
            jax.ShapeDtypeStruct((1, 1), jnp.float32),
        ],
        scratch_shapes=[pltpu.SMEM((3,), jnp.float32)],
        compiler_params=pltpu.CompilerParams(
            dimension_semantics=("arbitrary",)),
    )(uh, vh, bilin_W, tcode)


def kernel(u, v, u_table, v_table, gcl_W, gcl_b, dense_W, dense_b, bilin_W,
           ratings):
    u = u.astype(jnp.int32)
    v = v.astype(jnp.int32)
    u2 = u.reshape(_NW, _GR_W)
    v2 = jnp.concatenate([v, jnp.zeros((_BVP - _BV,), jnp.int32)]
                         ).reshape(_NW, _VE_W)

    code = _compress_call(ratings)
    gcode, uemb, vemb_p = _sc_gather(code, u2, v2, u_table, v_table)
    vemb = vemb_p[:_NV]
    v_row = v.reshape(1, _BV)

    bv, cnt = _prep_call(vemb, gcl_W, v_row)
    msgu, di, cs, tv, tcode = _agg_call(gcode, uemb, gcl_W, bv, cnt, v_row)
    uh, vh = _hidden_call(msgu, di, cs, tv, v_row, dense_W,
                          dense_b.reshape(1, _H1), gcl_b)
    mhat, loss, acc = _dec_call(uh, vh, bilin_W, tcode)
    return mhat, loss[0, 0], acc[0, 0]


# speedup vs baseline: 4.5219x; 1.0231x over previous
import functools

import jax
import jax.numpy as jnp
from jax import lax
from jax.experimental import pallas as pl
from jax.experimental.pallas import tpu as pltpu
from jax.experimental.pallas import tpu_sc as plsc

_R = 5
_NU = 10000
_NV = 1000
_D = 128
_H0 = 64
_H1 = 32
_BU = 4096
_BV = 1000
_KP = 1024

_NC = 2
_NS = 16
_NW = _NC * _NS
_GR_W = _BU // _NW
_CHUNK = 32
_NCHUNK = _GR_W // _CHUNK
_BVP = 1024
_VE_W = _BVP // _NW

_BUC = 400
_BLK = 256
_NB = _BU // _BLK


def _compress_body(ratings_ref, code_ref):
    acc = ratings_ref[0]
    for r in range(1, _R):
        acc = acc + float(r + 1) * ratings_ref[r]
    code_ref[...] = jnp.zeros((_BUC, _KP), jnp.float32)
    code_ref[:, : _NV] = acc


def _compress_call(ratings):
    n = _NU // _BUC
    return pl.pallas_call(
        _compress_body,
        grid=(n,),
        in_specs=[pl.BlockSpec((_R, _BUC, _NV), lambda i: (0, i, 0))],
        out_specs=pl.BlockSpec((_BUC, _KP), lambda i: (i, 0)),
        out_shape=jax.ShapeDtypeStruct((_NU, _KP), jnp.float32),
        compiler_params=pltpu.CompilerParams(
            dimension_semantics=("arbitrary",)),
    )(ratings)


def _sc_gather_body(code, u2, v2, u_table, v_table,
                    gc_out, ue_out, ve_out,
                    uidx, vidx, rows_a, rows_b, erows, vrows,
                    sem_a, sem_b, sem_e):
    wid = lax.axis_index("s") * _NC + lax.axis_index("c")
    base = wid * _GR_W

    pltpu.sync_copy(u2.at[wid], uidx)

    pltpu.async_copy(u_table.at[uidx], erows, sem_e).wait()
    pltpu.sync_copy(erows, ue_out.at[pl.ds(base, _GR_W)])

    pltpu.sync_copy(v2.at[wid], vidx)
    pltpu.async_copy(v_table.at[vidx], vrows, sem_e).wait()
    pltpu.sync_copy(vrows, ve_out.at[pl.ds(wid * _VE_W, _VE_W)])

    bufs = (rows_a, rows_b)
    sems = (sem_a, sem_b)
    handles = [None, None]
    handles[0] = pltpu.async_copy(
        code.at[uidx.at[pl.ds(0, _CHUNK)]], rows_a, sem_a)
    for c in range(_NCHUNK):
        if c + 1 < _NCHUNK:
            handles[(c + 1) % 2] = pltpu.async_copy(
                code.at[uidx.at[pl.ds((c + 1) * _CHUNK, _CHUNK)]],
                bufs[(c + 1) % 2], sems[(c + 1) % 2])
        handles[c % 2].wait()
        pltpu.sync_copy(bufs[c % 2],
                        gc_out.at[pl.ds(base + c * _CHUNK, _CHUNK)])


_sc_gather = functools.partial(
    pl.kernel,
    mesh=plsc.VectorSubcoreMesh(core_axis_name="c", subcore_axis_name="s"),
    out_type=[
        jax.ShapeDtypeStruct((_BU, _KP), jnp.float32),
        jax.ShapeDtypeStruct((_BU, _D), jnp.float32),
        jax.ShapeDtypeStruct((_BVP, _D), jnp.float32),
    ],
    scratch_types=[
        pltpu.VMEM((_GR_W,), jnp.int32),
        pltpu.VMEM((_VE_W,), jnp.int32),
        pltpu.VMEM((_CHUNK, _KP), jnp.float32),
        pltpu.VMEM((_CHUNK, _KP), jnp.float32),
        pltpu.VMEM((_GR_W, _D), jnp.float32),
        pltpu.VMEM((_VE_W, _D), jnp.float32),
        pltpu.SemaphoreType.DMA,
        pltpu.SemaphoreType.DMA,
        pltpu.SemaphoreType.DMA,
    ],
)(_sc_gather_body)


def _main_body(gc_ref, ue_ref, vemb_ref, gclW_ref, v_ref, dW_ref, db_ref,
               gclb_ref, bw_ref,
               mhat_ref, loss_ref, acc_ref,
               bv_s, cnt_s, msgu_s, di_s, cs_s, tv_s, tcode_s, uh_s, vh_s,
               sacc):
    i = pl.program_id(0)

    @pl.when(i == 0)
    def _prep():
        kio = lax.broadcasted_iota(jnp.int32, (_KP, _BV), 0)
        G = (kio == v_ref[...]).astype(jnp.float32)
        cnt_s[...] = jnp.sum(G, axis=1, keepdims=True)
        ve = vemb_ref[...]
        for r in range(_R):
            sv = jnp.dot(ve, gclW_ref[r], preferred_element_type=jnp.float32)
            bv_s[r] = jnp.dot(G, sv, preferred_element_type=jnp.float32)

    @pl.when(i < _NB)
    def _agg():
        ue = ue_ref[...]
        gc = gc_ref[...]
        acc_msg = jnp.zeros((_BLK, _H0), jnp.float32)
        acc_di = jnp.zeros((_BLK, 1), jnp.float32)
        acc_cs = jnp.zeros((1, _KP), jnp.float32)
        acc_tv = jnp.zeros((_KP, _H0), jnp.float32)
        cnt_col = cnt_s[...]
        for r in range(_R):
            Rr = (gc == float(r + 1)).astype(jnp.float32)
            su = jnp.dot(ue, gclW_ref[r], preferred_element_type=jnp.float32)
            acc_msg = acc_msg + jnp.dot(Rr, bv_s[r],
                                        preferred_element_type=jnp.float32)
            acc_di = acc_di + jnp.dot(Rr, cnt_col,
                                      preferred_element_type=jnp.float32)
            acc_cs = acc_cs + jnp.sum(Rr, axis=0, keepdims=True)
            acc_tv = acc_tv + lax.dot_general(
                Rr, su, (((0,), (0,)), ((), ())),
                preferred_element_type=jnp.float32)

        kio = lax.broadcasted_iota(jnp.int32, (_KP, _BV), 0)
        Gb = (kio == v_ref[...]).astype(jnp.bfloat16)
        tcode = lax.dot_general(
            gc.astype(jnp.bfloat16), Gb, (((1,), (0,)), ((), ())),
            preferred_element_type=jnp.float32)
        tcode_s[pl.ds(i * _BLK, _BLK)] = tcode.astype(jnp.bfloat16)
        msgu_s[pl.ds(i * _BLK, _BLK)] = acc_msg
        di_s[pl.ds(i * _BLK, _BLK)] = acc_di

        @pl.when(i == 0)
        def _():
            cs_s[...] = acc_cs
            tv_s[...] = acc_tv

        @pl.when(i != 0)
        def _():
            cs_s[...] = cs_s[...] + acc_cs
            tv_s[...] = tv_s[...] + acc_tv

    @pl.when(i >= _NB)
    def _decode():
        j = i - _NB

        @pl.when(j == 0)
        def _hidden():
            kio = lax.broadcasted_iota(jnp.int32, (_KP, _BV), 0)
            G = (kio == v_ref[...]).astype(jnp.float32)
            du = lax.dot_general(G, cs_s[...], (((0,), (1,)), ((), ())),
                                 preferred_element_type=jnp.float32)
            msgv = lax.dot_general(G, tv_s[...], (((0,), (0,)), ((), ())),
                                   preferred_element_type=jnp.float32)
            deg = jnp.concatenate([du, di_s[...]], axis=0)
            c = jnp.where(deg > 0, 1.0 / jnp.where(deg > 0, deg, 1.0), 0.0)
            cu = c[:_BU]
            ci = c[_BU:]
            bsum = jnp.sum(gclb_ref[...], axis=0, keepdims=True)
            zu = jnp.maximum(msgu_s[...] * cu + bsum, 0.0)
            zv = jnp.maximum(msgv * ci + bsum, 0.0)
            dW = dW_ref[...]
            db = db_ref[...]
            uh_s[...] = jax.nn.sigmoid(
                jnp.dot(zu, dW, preferred_element_type=jnp.float32) + db)
            vh_s[...] = jax.nn.sigmoid(
                jnp.dot(zv, dW, preferred_element_type=jnp.float32) + db)

        uh = uh_s[pl.ds(j * _BLK, _BLK)]
        vh = vh_s[...]
        Os = []
        for r in range(_R):
            A = jnp.dot(uh, bw_ref[r], preferred_element_type=jnp.float32)
            Os.append(lax.dot_general(A, vh, (((1,), (1,)), ((), ())),
                                      preferred_element_type=jnp.float32))
        mx = Os[0]
        for r in range(1, _R):
            mx = jnp.maximum(mx, Os[r])
        es = [jnp.exp(o - mx) for o in Os]
        se = es[0]
        for r in range(1, _R):
            se = se + es[r]
        num = jnp.zeros_like(se)
        for r in range(1, _R):
            num = num + float(r) * es[r]
        mhat_ref[...] = num / se

        tc = tcode_s[pl.ds(j * _BLK, _BLK)].astype(jnp.float32)
        obs = tc > 0.5
        ot = jnp.zeros_like(mx)
        for r in range(_R):
            ot = jnp.where(tc == float(r + 1), Os[r], ot)
        lterm = jnp.where(obs, mx + jnp.log(se) - ot, 0.0)

        pbest = Os[0]
        pcls = jnp.zeros_like(mx)
        for r in range(1, _R):
            gt = Os[r] > pbest
            pbest = jnp.where(gt, Os[r], pbest)
            pcls = jnp.where(gt, float(r), pcls)
        corr = jnp.where(obs & (pcls == (tc - 1.0)), 1.0, 0.0)

        ls = jnp.sum(lterm)
        nb = jnp.sum(jnp.where(obs, 1.0, 0.0))
        cr = jnp.sum(corr)

        @pl.when(j == 0)
        def _():
            sacc[0] = ls
            sacc[1] = nb
            sacc[2] = cr

        @pl.when(j != 0)
        def _():
            sacc[0] = sacc[0] + ls
            sacc[1] = sacc[1] + nb
            sacc[2] = sacc[2] + cr

        @pl.when(j == _NB - 1)
        def _():
            nbm = jnp.maximum(sacc[1], 1.0)
            loss_ref[...] = jnp.broadcast_to(sacc[0] / nbm, (1, 1))
            acc_ref[...] = jnp.broadcast_to(sacc[2] / nbm, (1, 1))


def _main_call(gcode, uemb, vemb, gcl_W, v_row, dense_W, db_row, gcl_b,
               bilin_W):
    return pl.pallas_call(
        _main_body,
        grid=(2 * _NB,),
        in_specs=[
            pl.BlockSpec((_BLK, _KP), lambda i: (jnp.minimum(i, _NB - 1), 0)),
            pl.BlockSpec((_BLK, _D), lambda i: (jnp.minimum(i, _NB - 1), 0)),
            pl.BlockSpec((_NV, _D), lambda i: (0, 0)),
            pl.BlockSpec((_R, _D, _H0), lambda i: (0, 0, 0)),
            pl.BlockSpec((1, _BV), lambda i: (0, 0)),
            pl.BlockSpec((_H0, _H1), lambda i: (0, 0)),
            pl.BlockSpec((1, _H1), lambda i: (0, 0)),
            pl.BlockSpec((_R, _H0), lambda i: (0, 0)),
            pl.BlockSpec((_R, _H1, _H1), lambda i: (0, 0, 0)),
        ],
        out_specs=[
            pl.BlockSpec((_BLK, _BV), lambda i: (jnp.maximum(i - _NB, 0), 0)),
            pl.BlockSpec((1, 1), lambda i: (0, 0)),
            pl.BlockSpec((1, 1), lambda i: (0, 0)),
        ],
        out_shape=[
            jax.ShapeDtypeStruct((_BU, _BV), jnp.float32),
            jax.ShapeDtypeStruct((1, 1), jnp.float32),
            jax.ShapeDtypeStruct((1, 1), jnp.float32),
        ],
        scratch_shapes=[
            pltpu.VMEM((_R, _KP, _H0), jnp.float32),
            pltpu.VMEM((_KP, 1), jnp.float32),
            pltpu.VMEM((_BU, _H0), jnp.float32),
            pltpu.VMEM((_BU, 1), jnp.float32),
            pltpu.VMEM((1, _KP), jnp.float32),
            pltpu.VMEM((_KP, _H0), jnp.float32),
            pltpu.VMEM((_BU, _BV), jnp.bfloat16),
            pltpu.VMEM((_BU, _H1), jnp.float32),
            pltpu.VMEM((_NV, _H1), jnp.float32),
            pltpu.SMEM((3,), jnp.float32),
        ],
        compiler_params=pltpu.CompilerParams(
            dimension_semantics=("arbitrary",)),
    )(gcode, uemb, vemb, gcl_W, v_row, dense_W, db_row, gcl_b, bilin_W)


def kernel(u, v, u_table, v_table, gcl_W, gcl_b, dense_W, dense_b, bilin_W,
           ratings):
    u = u.astype(jnp.int32)
    v = v.astype(jnp.int32)
    u2 = u.reshape(_NW, _GR_W)
    v2 = jnp.concatenate([v, jnp.zeros((_BVP - _BV,), jnp.int32)]
                         ).reshape(_NW, _VE_W)

    code = _compress_call(ratings)
    gcode, uemb, vemb_p = _sc_gather(code, u2, v2, u_table, v_table)
    vemb = vemb_p[:_NV]
    v_row = v.reshape(1, _BV)

    mhat, loss, acc = _main_call(gcode, uemb, vemb, gcl_W, v_row, dense_W,
                                 dense_b.reshape(1, _H1), gcl_b, bilin_W)
    return mhat, loss[0, 0], acc[0, 0]
